# initial kernel scaffold (unmeasured)
import jax
import jax.numpy as jnp
from jax import lax
from jax.experimental import pallas as pl
from jax.experimental.pallas import tpu as pltpu

N_DEV = 16
M_PER = 256
N_PER = 128
K = 4096


def kernel(x, w_mat):
    def body(x_ref, w_ref, out_ref, send_buf, comm_ref, send_sems, recv_sems):
        my_pos = lax.axis_index("i")

        xb = x_ref[...].astype(jnp.bfloat16)
        for j in range(N_DEV):
            wb = w_ref[:, j * N_PER:(j + 1) * N_PER].astype(jnp.bfloat16)
            piece = jnp.dot(xb, wb, preferred_element_type=jnp.float32)
            send_buf[j] = piece.astype(jnp.bfloat16)

        for j in range(N_DEV):
            @pl.when(j != my_pos)
            def _():
                rdma = pltpu.make_async_remote_copy(
                    src_ref=send_buf.at[j],
                    dst_ref=comm_ref.at[my_pos],
                    send_sem=send_sems.at[j],
                    recv_sem=recv_sems.at[my_pos],
                    device_id=(j,),
                    device_id_type=pl.DeviceIdType.MESH,
                )
                rdma.start()

        comm_ref[my_pos] = send_buf[my_pos]

        for i in range(N_DEV):
            @pl.when(i != my_pos)
            def _():
                recv = pltpu.make_async_remote_copy(
                    src_ref=send_buf.at[i],
                    dst_ref=comm_ref.at[i],
                    send_sem=send_sems.at[i],
                    recv_sem=recv_sems.at[i],
                    device_id=(i,),
                    device_id_type=pl.DeviceIdType.MESH,
                )
                recv.wait_recv()
            out_ref[pl.ds(i * M_PER, M_PER), :] = comm_ref[i].astype(jnp.float32)

        for j in range(N_DEV):
            @pl.when(j != my_pos)
            def _():
                send = pltpu.make_async_remote_copy(
                    src_ref=send_buf.at[j],
                    dst_ref=comm_ref.at[my_pos],
                    send_sem=send_sems.at[j],
                    recv_sem=recv_sems.at[my_pos],
                    device_id=(j,),
                    device_id_type=pl.DeviceIdType.MESH,
                )
                send.wait_send()

    return pl.pallas_call(
        body,
        out_shape=jax.ShapeDtypeStruct((N_DEV * M_PER, N_PER), jnp.float32),
        in_specs=[
            pl.BlockSpec(memory_space=pltpu.VMEM),
            pl.BlockSpec(memory_space=pltpu.VMEM),
        ],
        out_specs=pl.BlockSpec(memory_space=pltpu.VMEM),
        scratch_shapes=[
            pltpu.VMEM((N_DEV, M_PER, N_PER), jnp.bfloat16),
            pltpu.VMEM((N_DEV, M_PER, N_PER), jnp.bfloat16),
            pltpu.SemaphoreType.DMA((N_DEV,)),
            pltpu.SemaphoreType.DMA((N_DEV,)),
        ],
    )(x, w_mat)


# baseline (device time: 45942 ns/iter reference)
import jax
import jax.numpy as jnp
from jax import lax
from jax.experimental import pallas as pl
from jax.experimental.pallas import tpu as pltpu

N_DEV = 16
M_PER = 256
N_PER = 128
K = 4096


def kernel(x, w_mat):
    def body(x_ref, w_ref, out_ref, send_buf, comm_ref, send_sems, recv_sems):
        my_pos = lax.axis_index("i")

        xb = x_ref[...].astype(jnp.bfloat16)
        for j in range(N_DEV):
            wb = w_ref[:, j * N_PER:(j + 1) * N_PER].astype(jnp.bfloat16)
            piece = jnp.dot(xb, wb, preferred_element_type=jnp.float32)
            send_buf[j] = piece.astype(jnp.bfloat16)

        for j in range(N_DEV):
            @pl.when(j != my_pos)
            def _():
                rdma = pltpu.make_async_remote_copy(
                    src_ref=send_buf.at[j],
                    dst_ref=comm_ref.at[my_pos],
                    send_sem=send_sems.at[j],
                    recv_sem=recv_sems.at[my_pos],
                    device_id=(j,),
                    device_id_type=pl.DeviceIdType.MESH,
                )
                rdma.start()

        comm_ref[my_pos] = send_buf[my_pos]

        for i in range(N_DEV):
            @pl.when(i != my_pos)
            def _():
                recv = pltpu.make_async_remote_copy(
                    src_ref=send_buf.at[i],
                    dst_ref=comm_ref.at[i],
                    send_sem=send_sems.at[i],
                    recv_sem=recv_sems.at[i],
                    device_id=(i,),
                    device_id_type=pl.DeviceIdType.MESH,
                )
                recv.wait_recv()
            out_ref[pl.ds(i * M_PER, M_PER), :] = comm_ref[i].astype(jnp.float32)

        for j in range(N_DEV):
            @pl.when(j != my_pos)
            def _():
                send = pltpu.make_async_remote_copy(
                    src_ref=send_buf.at[j],
                    dst_ref=comm_ref.at[my_pos],
                    send_sem=send_sems.at[j],
                    recv_sem=recv_sems.at[my_pos],
                    device_id=(j,),
                    device_id_type=pl.DeviceIdType.MESH,
                )
                send.wait_send()

    return pl.pallas_call(
        body,
        out_shape=jax.ShapeDtypeStruct((N_DEV * M_PER, N_PER), jnp.float32),
        in_specs=[
            pl.BlockSpec(memory_space=pltpu.VMEM),
            pl.BlockSpec(memory_space=pltpu.VMEM),
        ],
        out_specs=pl.BlockSpec(memory_space=pltpu.VMEM),
        scratch_shapes=[
            pltpu.VMEM((N_DEV, M_PER, N_PER), jnp.bfloat16),
            pltpu.VMEM((N_DEV, M_PER, N_PER), jnp.bfloat16),
            pltpu.SemaphoreType.DMA((N_DEV,)),
            pltpu.SemaphoreType.DMA((N_DEV,)),
        ],
        compiler_params=pltpu.CompilerParams(
            vmem_limit_bytes=56 * 1024 * 1024,
        ),
    )(x, w_mat)


# device time: 33330 ns/iter; 1.3784x vs baseline; 1.3784x over previous
import jax
import jax.numpy as jnp
from jax import lax
from jax.experimental import pallas as pl
from jax.experimental.pallas import tpu as pltpu

N_DEV = 16
M_PER = 256
N_PER = 128
K = 4096
N_SLOTS = 4


def kernel(x, w_mat):
    def body(x_ref, w_ref, out_ref, w_tiles, send_buf, comm_ref,
             load_sems, send_sems, recv_sems):
        my_pos = lax.axis_index("i")

        def tile_of(t):
            return lax.rem(my_pos + 1 + t, N_DEV)

        def start_load(t):
            j = tile_of(t)
            pltpu.make_async_copy(
                w_ref.at[:, pl.ds(j * N_PER, N_PER)],
                w_tiles.at[t % N_SLOTS],
                load_sems.at[t % N_SLOTS],
            ).start()

        for t in range(N_SLOTS - 1):
            start_load(t)

        xb = x_ref[...].astype(jnp.bfloat16)

        for t in range(N_DEV):
            if t + N_SLOTS - 1 < N_DEV:
                start_load(t + N_SLOTS - 1)
            slot = t % N_SLOTS
            pltpu.make_async_copy(
                w_ref.at[:, pl.ds(tile_of(t) * N_PER, N_PER)],
                w_tiles.at[slot],
                load_sems.at[slot],
            ).wait()

            j = tile_of(t)
            wb = w_tiles[slot].astype(jnp.bfloat16)
            piece = jnp.dot(xb, wb, preferred_element_type=jnp.float32)
            send_buf[j] = piece.astype(jnp.bfloat16)

            if t < N_DEV - 1:
                pltpu.make_async_remote_copy(
                    src_ref=send_buf.at[j],
                    dst_ref=comm_ref.at[my_pos],
                    send_sem=send_sems.at[j],
                    recv_sem=recv_sems.at[my_pos],
                    device_id=(j,),
                    device_id_type=pl.DeviceIdType.MESH,
                ).start()
            else:
                comm_ref[my_pos] = send_buf[my_pos]

        out_ref[pl.ds(my_pos * M_PER, M_PER), :] = (
            comm_ref[my_pos].astype(jnp.float32))

        for t in range(N_DEV - 1):
            i = lax.rem(my_pos - 1 - t + 2 * N_DEV, N_DEV)
            pltpu.make_async_remote_copy(
                src_ref=send_buf.at[i],
                dst_ref=comm_ref.at[i],
                send_sem=send_sems.at[i],
                recv_sem=recv_sems.at[i],
                device_id=(i,),
                device_id_type=pl.DeviceIdType.MESH,
            ).wait_recv()
            out_ref[pl.ds(i * M_PER, M_PER), :] = comm_ref[i].astype(jnp.float32)

        for t in range(N_DEV - 1):
            j = tile_of(t)
            pltpu.make_async_remote_copy(
                src_ref=send_buf.at[j],
                dst_ref=comm_ref.at[my_pos],
                send_sem=send_sems.at[j],
                recv_sem=recv_sems.at[my_pos],
                device_id=(j,),
                device_id_type=pl.DeviceIdType.MESH,
            ).wait_send()

    return pl.pallas_call(
        body,
        out_shape=jax.ShapeDtypeStruct((N_DEV * M_PER, N_PER), jnp.float32),
        in_specs=[
            pl.BlockSpec(memory_space=pltpu.VMEM),
            pl.BlockSpec(memory_space=pl.ANY),
        ],
        out_specs=pl.BlockSpec(memory_space=pltpu.VMEM),
        scratch_shapes=[
            pltpu.VMEM((N_SLOTS, K, N_PER), jnp.float32),
            pltpu.VMEM((N_DEV, M_PER, N_PER), jnp.bfloat16),
            pltpu.VMEM((N_DEV, M_PER, N_PER), jnp.bfloat16),
            pltpu.SemaphoreType.DMA((N_SLOTS,)),
            pltpu.SemaphoreType.DMA((N_DEV,)),
            pltpu.SemaphoreType.DMA((N_DEV,)),
        ],
        compiler_params=pltpu.CompilerParams(
            vmem_limit_bytes=56 * 1024 * 1024,
        ),
    )(x, w_mat)


# device time: 27155 ns/iter; 1.6918x vs baseline; 1.2274x over previous
import jax
import jax.numpy as jnp
from jax import lax
from jax.experimental import pallas as pl
from jax.experimental.pallas import tpu as pltpu

N_DEV = 16
M_PER = 256
N_PER = 128
K = 4096
N_SLOTS = 6


def kernel(x, w_mat):
    def body(x_ref, w_ref, out_ref, x_vmem, w_tiles, send_buf,
             x_sem, load_sems, send_sems, recv_sems, entry_sems):
        my_pos = lax.axis_index("i")

        barrier_sem = pltpu.get_barrier_semaphore()
        pl.semaphore_signal(barrier_sem, inc=1)
        pl.semaphore_wait(barrier_sem, 1)
        for nbr in range(N_DEV):
            @pl.when(nbr != my_pos)
            def _():
                pl.semaphore_signal(
                    entry_sems.at[my_pos], inc=1,
                    device_id=(nbr,), device_id_type=pl.DeviceIdType.MESH,
                )

        def tile_of(t):
            return lax.rem(my_pos + 1 + t, N_DEV)

        def start_load(t):
            pltpu.make_async_copy(
                w_ref.at[:, pl.ds(tile_of(t) * N_PER, N_PER)],
                w_tiles.at[t % N_SLOTS],
                load_sems.at[t % N_SLOTS],
            ).start()

        my_rows = pl.ds(my_pos * M_PER, M_PER)

        x_dma = pltpu.make_async_copy(x_ref, x_vmem, x_sem)
        x_dma.start()
        for t in range(N_SLOTS - 1):
            start_load(t)
        x_dma.wait()
        xb = x_vmem[...].astype(jnp.bfloat16)

        for t in range(N_DEV):
            if t + N_SLOTS - 1 < N_DEV:
                start_load(t + N_SLOTS - 1)
            slot = t % N_SLOTS
            pltpu.make_async_copy(
                w_ref.at[:, pl.ds(tile_of(t) * N_PER, N_PER)],
                w_tiles.at[slot],
                load_sems.at[slot],
            ).wait()

            j = tile_of(t)
            wb = w_tiles[slot].astype(jnp.bfloat16)
            piece = jnp.dot(xb, wb, preferred_element_type=jnp.float32)

            if t < N_DEV - 1:
                send_buf[j] = piece.astype(jnp.bfloat16)
                pl.semaphore_wait(entry_sems.at[j], 1)
                pltpu.make_async_remote_copy(
                    src_ref=send_buf.at[j],
                    dst_ref=out_ref.at[my_rows, :],
                    send_sem=send_sems.at[j],
                    recv_sem=recv_sems.at[my_pos],
                    device_id=(j,),
                    device_id_type=pl.DeviceIdType.MESH,
                ).start()
            else:
                out_ref[my_rows, :] = piece.astype(jnp.bfloat16)

        for i in range(N_DEV):
            @pl.when(i != my_pos)
            def _():
                pltpu.make_async_remote_copy(
                    src_ref=send_buf.at[i],
                    dst_ref=out_ref.at[pl.ds(i * M_PER, M_PER), :],
                    send_sem=send_sems.at[i],
                    recv_sem=recv_sems.at[i],
                    device_id=(i,),
                    device_id_type=pl.DeviceIdType.MESH,
                ).wait_recv()

        for t in range(N_DEV - 1):
            j = tile_of(t)
            pltpu.make_async_remote_copy(
                src_ref=send_buf.at[j],
                dst_ref=out_ref.at[my_rows, :],
                send_sem=send_sems.at[j],
                recv_sem=recv_sems.at[my_pos],
                device_id=(j,),
                device_id_type=pl.DeviceIdType.MESH,
            ).wait_send()

    return pl.pallas_call(
        body,
        out_shape=jax.ShapeDtypeStruct((N_DEV * M_PER, N_PER), jnp.bfloat16),
        in_specs=[
            pl.BlockSpec(memory_space=pl.ANY),
            pl.BlockSpec(memory_space=pl.ANY),
        ],
        out_specs=pl.BlockSpec(memory_space=pltpu.VMEM),
        scratch_shapes=[
            pltpu.VMEM((M_PER, K), jnp.float32),
            pltpu.VMEM((N_SLOTS, K, N_PER), jnp.float32),
            pltpu.VMEM((N_DEV, M_PER, N_PER), jnp.bfloat16),
            pltpu.SemaphoreType.DMA,
            pltpu.SemaphoreType.DMA((N_SLOTS,)),
            pltpu.SemaphoreType.DMA((N_DEV,)),
            pltpu.SemaphoreType.DMA((N_DEV,)),
            pltpu.SemaphoreType.REGULAR((N_DEV,)),
        ],
        compiler_params=pltpu.CompilerParams(
            vmem_limit_bytes=56 * 1024 * 1024,
            collective_id=0,
        ),
    )(x, w_mat)
